# baseline (speedup 1.0000x reference)
import functools

import jax
import jax.numpy as jnp
from jax import lax
from jax.experimental import pallas as pl
from jax.experimental.pallas import tpu as pltpu
from jax.experimental.pallas import tpu_sc as plsc

_PADDING_IDX = 0
_SMOOTHING = 0.1
_CONFIDENCE = 1.0 - _SMOOTHING


def _sc_gather_target_rows(x2, target_flat):
    n = target_flat.shape[0]
    info = plsc.get_sparse_core_info()
    nc, ns = info.num_cores, info.num_subcores
    nw = nc * ns
    n_per_w = n // nw
    mesh = plsc.VectorSubcoreMesh(core_axis_name="c", subcore_axis_name="s")
    chunk = 32

    @functools.partial(
        pl.kernel,
        mesh=mesh,
        out_type=jax.ShapeDtypeStruct((n, 128), jnp.float32),
        scratch_types=[
            pltpu.VMEM((n_per_w,), jnp.int32),
            pltpu.VMEM((2, chunk, 8, 128), jnp.float32),
            pltpu.VMEM((n_per_w, 128), jnp.float32),
            pltpu.SemaphoreType.DMA,
            pltpu.SemaphoreType.DMA,
        ],
    )
    def gather_kernel(x_hbm, tgt_hbm, out_hbm, t_v, tiles_v, rows_v, sem0, sem1):
        wid = lax.axis_index("s") * nc + lax.axis_index("c")
        base = wid * n_per_w
        pltpu.sync_copy(tgt_hbm.at[pl.ds(base, n_per_w)], t_v)
        sems = (sem0, sem1)

        def fire(k):
            copies = []
            for jo in range(chunk // 16):
                t_vec = t_v[pl.ds(k * chunk + jo * 16, 16)]
                cbs = lax.shift_left(lax.shift_right_logical(t_vec, 7), 7)
                for ji in range(16):
                    j = k * chunk + jo * 16 + ji
                    copies.append(
                        pltpu.async_copy(
                            x_hbm.at[
                                pl.ds(base + (j & ~7), 8),
                                pl.ds(pl.multiple_of(cbs[ji], 128), 128),
                            ],
                            tiles_v.at[k & 1, j - k * chunk],
                            sems[k & 1],
                        )
                    )
            return copies

        pending = fire(0)
        for k in range(n_per_w // chunk):
            nxt = fire(k + 1) if (k + 1) < n_per_w // chunk else []
            for c in pending:
                c.wait()
            pending = nxt
            for jj in range(chunk):
                j = k * chunk + jj
                for c8 in range(8):
                    rows_v[j, pl.ds(c8 * 16, 16)] = tiles_v[k & 1, jj, j & 7, pl.ds(c8 * 16, 16)]
        pltpu.sync_copy(rows_v, out_hbm.at[pl.ds(base, n_per_w)])

    return gather_kernel(x2, target_flat)


def _tc_rowsum_loss(x2, tgt3, vocab, block_rows, block_vocab):
    rows = x2.shape[0]
    gr = rows // block_rows
    gv = vocab // block_vocab
    e_k = _SMOOTHING / (vocab - 2)

    def body(x_ref, t_ref, out_ref):
        i = pl.program_id(0)
        j = pl.program_id(1)
        rs = jnp.sum(x_ref[...], axis=1)
        t = t_ref[0, 0, :]
        per = jnp.where(t != _PADDING_IDX, rs, 0.0)

        @pl.when(jnp.logical_and(i == 0, j == 0))
        def _():
            out_ref[...] = jnp.zeros_like(out_ref)

        out_ref[...] += jnp.reshape(-e_k * jnp.sum(per), (1, 1))

    out = pl.pallas_call(
        body,
        grid=(gr, gv),
        in_specs=[
            pl.BlockSpec((block_rows, block_vocab), lambda i, j: (i, j)),
            pl.BlockSpec((1, 1, block_rows), lambda i, j: (i, 0, 0)),
        ],
        out_specs=pl.BlockSpec((1, 1), lambda i, j: (0, 0)),
        out_shape=jax.ShapeDtypeStruct((1, 1), jnp.float32),
    )(x2, tgt3)
    return out[0, 0]


def _tc_gather_loss(tgt3, rows3, vocab, block_rows):
    n_blocks = rows3.shape[0]
    e_k = _SMOOTHING / (vocab - 2)

    def body(t_ref, r_ref, out_ref):
        i = pl.program_id(0)
        t = t_ref[0, 0, :]
        gr = r_ref[0, :, :]
        c = lax.bitwise_and(t, 127)
        lane = lax.broadcasted_iota(jnp.int32, (block_rows, 128), 1)
        g = jnp.sum(jnp.where(lane == c[:, None], gr, 0.0), axis=1)
        per = jnp.where(t != _PADDING_IDX, g, 0.0)

        @pl.when(i == 0)
        def _():
            out_ref[...] = jnp.zeros_like(out_ref)

        out_ref[...] += jnp.reshape(-(_CONFIDENCE - e_k) * jnp.sum(per), (1, 1))

    out = pl.pallas_call(
        body,
        grid=(n_blocks,),
        in_specs=[
            pl.BlockSpec((1, 1, block_rows), lambda i: (i, 0, 0)),
            pl.BlockSpec((1, block_rows, 128), lambda i: (i, 0, 0)),
        ],
        out_specs=pl.BlockSpec((1, 1), lambda i: (0, 0)),
        out_shape=jax.ShapeDtypeStruct((1, 1), jnp.float32),
    )(tgt3, rows3)
    return out[0, 0]


def kernel(x, target):
    b, l, v = x.shape
    r = b * l
    block_rows = 256
    block_vocab = 2048
    x2 = x.reshape(r, v)
    tflat = target.reshape(r)
    tgt3 = tflat.reshape(r // block_rows, 1, block_rows)
    grows = _sc_gather_target_rows(x2, tflat)
    s1 = _tc_rowsum_loss(x2, tgt3, v, block_rows, block_vocab)
    s2 = _tc_gather_loss(tgt3, grows.reshape(r // block_rows, block_rows, 128), v, block_rows)
    return s1 + s2


# speedup vs baseline: 1.0314x; 1.0314x over previous
import functools

import jax
import jax.numpy as jnp
from jax import lax
from jax.experimental import pallas as pl
from jax.experimental.pallas import tpu as pltpu
from jax.experimental.pallas import tpu_sc as plsc

_PADDING_IDX = 0
_SMOOTHING = 0.1
_CONFIDENCE = 1.0 - _SMOOTHING


def _sc_gather_target_rows(x2, target_flat):
    n = target_flat.shape[0]
    info = plsc.get_sparse_core_info()
    nc, ns = info.num_cores, info.num_subcores
    nw = nc * ns
    n_per_w = n // nw
    mesh = plsc.VectorSubcoreMesh(core_axis_name="c", subcore_axis_name="s")
    chunk = 32

    @functools.partial(
        pl.kernel,
        mesh=mesh,
        out_type=jax.ShapeDtypeStruct((n, 128), jnp.float32),
        scratch_types=[
            pltpu.VMEM((n_per_w,), jnp.int32),
            pltpu.VMEM((2, chunk, 8, 128), jnp.float32),
            pltpu.VMEM((n_per_w, 128), jnp.float32),
            pltpu.SemaphoreType.DMA,
            pltpu.SemaphoreType.DMA,
        ],
    )
    def gather_kernel(x_hbm, tgt_hbm, out_hbm, t_v, tiles_v, rows_v, sem0, sem1):
        wid = lax.axis_index("s") * nc + lax.axis_index("c")
        base = wid * n_per_w
        pltpu.sync_copy(tgt_hbm.at[pl.ds(base, n_per_w)], t_v)
        sems = (sem0, sem1)

        def fire(k):
            copies = []
            for jo in range(chunk // 16):
                t_vec = t_v[pl.ds(k * chunk + jo * 16, 16)]
                cbs = lax.shift_left(lax.shift_right_logical(t_vec, 7), 7)
                for ji in range(16):
                    j = k * chunk + jo * 16 + ji
                    copies.append(
                        pltpu.async_copy(
                            x_hbm.at[
                                pl.ds(base + (j & ~7), 8),
                                pl.ds(pl.multiple_of(cbs[ji], 128), 128),
                            ],
                            tiles_v.at[k & 1, j - k * chunk],
                            sems[k & 1],
                        )
                    )
            return copies

        pending = fire(0)
        for k in range(n_per_w // chunk):
            nxt = fire(k + 1) if (k + 1) < n_per_w // chunk else []
            for c in pending:
                c.wait()
            pending = nxt
            for jj in range(chunk):
                j = k * chunk + jj
                for c8 in range(8):
                    rows_v[j, pl.ds(c8 * 16, 16)] = tiles_v[k & 1, jj, j & 7, pl.ds(c8 * 16, 16)]
        pltpu.sync_copy(rows_v, out_hbm.at[pl.ds(base, n_per_w)])

    return gather_kernel(x2, target_flat)


def _tc_rowsum_loss(x2, tgt3, vocab, block_rows, block_vocab):
    rows = x2.shape[0]
    gr = rows // block_rows
    gv = vocab // block_vocab
    e_k = _SMOOTHING / (vocab - 2)

    def body(x_ref, t_ref, out_ref):
        i = pl.program_id(0)
        j = pl.program_id(1)
        rs = jnp.sum(x_ref[...], axis=1)
        t = t_ref[0, 0, :]
        per = jnp.where(t != _PADDING_IDX, rs, 0.0)

        @pl.when(jnp.logical_and(i == 0, j == 0))
        def _():
            out_ref[...] = jnp.zeros_like(out_ref)

        out_ref[...] += jnp.reshape(-e_k * jnp.sum(per), (1, 1))

    out = pl.pallas_call(
        body,
        grid=(gr, gv),
        in_specs=[
            pl.BlockSpec((block_rows, block_vocab), lambda i, j: (i, j)),
            pl.BlockSpec((1, 1, block_rows), lambda i, j: (i, 0, 0)),
        ],
        out_specs=pl.BlockSpec((1, 1), lambda i, j: (0, 0)),
        out_shape=jax.ShapeDtypeStruct((1, 1), jnp.float32),
    )(x2, tgt3)
    return out[0, 0]


def _tc_gather_loss(tgt3, rows3, vocab, block_rows):
    n_blocks = rows3.shape[0]
    e_k = _SMOOTHING / (vocab - 2)

    def body(t_ref, r_ref, out_ref):
        i = pl.program_id(0)
        t = t_ref[0, 0, :]
        gr = r_ref[0, :, :]
        c = lax.bitwise_and(t, 127)
        lane = lax.broadcasted_iota(jnp.int32, (block_rows, 128), 1)
        g = jnp.sum(jnp.where(lane == c[:, None], gr, 0.0), axis=1)
        per = jnp.where(t != _PADDING_IDX, g, 0.0)

        @pl.when(i == 0)
        def _():
            out_ref[...] = jnp.zeros_like(out_ref)

        out_ref[...] += jnp.reshape(-(_CONFIDENCE - e_k) * jnp.sum(per), (1, 1))

    out = pl.pallas_call(
        body,
        grid=(n_blocks,),
        in_specs=[
            pl.BlockSpec((1, 1, block_rows), lambda i: (i, 0, 0)),
            pl.BlockSpec((1, block_rows, 128), lambda i: (i, 0, 0)),
        ],
        out_specs=pl.BlockSpec((1, 1), lambda i: (0, 0)),
        out_shape=jax.ShapeDtypeStruct((1, 1), jnp.float32),
    )(tgt3, rows3)
    return out[0, 0]


def kernel(x, target):
    b, l, v = x.shape
    r = b * l
    block_rows = 128
    block_vocab = 8192
    x2 = x.reshape(r, v)
    tflat = target.reshape(r)
    tgt3 = tflat.reshape(r // block_rows, 1, block_rows)
    grows = _sc_gather_target_rows(x2, tflat)
    s1 = _tc_rowsum_loss(x2, tgt3, v, block_rows, block_vocab)
    s2 = _tc_gather_loss(tgt3, grows.reshape(r // block_rows, block_rows, 128), v, block_rows)
    return s1 + s2


# speedup vs baseline: 1.3328x; 1.2922x over previous
import functools

import jax
import jax.numpy as jnp
from jax import lax
from jax.experimental import pallas as pl
from jax.experimental.pallas import tpu as pltpu
from jax.experimental.pallas import tpu_sc as plsc

_PADDING_IDX = 0
_SMOOTHING = 0.1
_CONFIDENCE = 1.0 - _SMOOTHING


def _sc_gather_target_rows(x2, target_flat):
    n = target_flat.shape[0]
    info = plsc.get_sparse_core_info()
    nc, ns = info.num_cores, info.num_subcores
    nw = nc * ns
    n_per_w = n // nw
    mesh = plsc.VectorSubcoreMesh(core_axis_name="c", subcore_axis_name="s")
    chunk = 32

    @functools.partial(
        pl.kernel,
        mesh=mesh,
        out_type=jax.ShapeDtypeStruct((n, 128), jnp.float32),
        scratch_types=[
            pltpu.VMEM((n_per_w,), jnp.int32),
            pltpu.VMEM((2, chunk, 8, 128), jnp.float32),
            pltpu.VMEM((n_per_w, 128), jnp.float32),
            pltpu.SemaphoreType.DMA,
            pltpu.SemaphoreType.DMA,
        ],
    )
    def gather_kernel(x_hbm, tgt_hbm, out_hbm, t_v, tiles_v, rows_v, sem0, sem1):
        wid = lax.axis_index("s") * nc + lax.axis_index("c")
        base = wid * n_per_w
        pltpu.sync_copy(tgt_hbm.at[pl.ds(base, n_per_w)], t_v)
        sems = (sem0, sem1)

        def fire(k):
            copies = []
            for jo in range(chunk // 16):
                t_vec = t_v[pl.ds(k * chunk + jo * 16, 16)]
                cbs = lax.shift_left(lax.shift_right_logical(t_vec, 7), 7)
                for ji in range(16):
                    j = k * chunk + jo * 16 + ji
                    copies.append(
                        pltpu.async_copy(
                            x_hbm.at[
                                pl.ds(base + (j & ~7), 8),
                                pl.ds(pl.multiple_of(cbs[ji], 128), 128),
                            ],
                            tiles_v.at[k & 1, j - k * chunk],
                            sems[k & 1],
                        )
                    )
            return copies

        pending = fire(0)
        for k in range(n_per_w // chunk):
            nxt = fire(k + 1) if (k + 1) < n_per_w // chunk else []
            for c in pending:
                c.wait()
            pending = nxt
            for jj in range(chunk):
                j = k * chunk + jj
                for c8 in range(8):
                    rows_v[j, pl.ds(c8 * 16, 16)] = tiles_v[k & 1, jj, j & 7, pl.ds(c8 * 16, 16)]
        pltpu.sync_copy(rows_v, out_hbm.at[pl.ds(base, n_per_w)])

    return gather_kernel(x2, target_flat)


def _tc_rowsum_loss(x2, tgt3, vocab, block_rows, block_vocab):
    rows = x2.shape[0]
    gr = rows // block_rows
    gv = vocab // block_vocab
    e_k = _SMOOTHING / (vocab - 2)

    def body(x_ref, t_ref, out_ref):
        i = pl.program_id(0)
        j = pl.program_id(1)
        rs = jnp.sum(x_ref[...], axis=1)
        t = t_ref[0, 0, :]
        per = jnp.where(t != _PADDING_IDX, rs, 0.0)

        @pl.when(jnp.logical_and(i == 0, j == 0))
        def _():
            out_ref[...] = jnp.zeros_like(out_ref)

        out_ref[...] += jnp.reshape(-e_k * jnp.sum(per), (1, 1))

    out = pl.pallas_call(
        body,
        grid=(gr, gv),
        in_specs=[
            pl.BlockSpec((block_rows, block_vocab), lambda i, j: (i, j)),
            pl.BlockSpec((1, 1, block_rows), lambda i, j: (i, 0, 0)),
        ],
        out_specs=pl.BlockSpec((1, 1), lambda i, j: (0, 0)),
        out_shape=jax.ShapeDtypeStruct((1, 1), jnp.float32),
    )(x2, tgt3)
    return out[0, 0]


def _tc_gather_loss(tgt3, rows3, vocab, block_rows):
    n_blocks = rows3.shape[0]
    e_k = _SMOOTHING / (vocab - 2)

    def body(t_ref, r_ref, out_ref):
        i = pl.program_id(0)
        t = t_ref[0, 0, :]
        gr = r_ref[0, :, :]
        c = lax.bitwise_and(t, 127)
        lane = lax.broadcasted_iota(jnp.int32, (block_rows, 128), 1)
        g = jnp.sum(jnp.where(lane == c[:, None], gr, 0.0), axis=1)
        per = jnp.where(t != _PADDING_IDX, g, 0.0)

        @pl.when(i == 0)
        def _():
            out_ref[...] = jnp.zeros_like(out_ref)

        out_ref[...] += jnp.reshape(-(_CONFIDENCE - e_k) * jnp.sum(per), (1, 1))

    out = pl.pallas_call(
        body,
        grid=(n_blocks,),
        in_specs=[
            pl.BlockSpec((1, 1, block_rows), lambda i: (i, 0, 0)),
            pl.BlockSpec((1, block_rows, 128), lambda i: (i, 0, 0)),
        ],
        out_specs=pl.BlockSpec((1, 1), lambda i: (0, 0)),
        out_shape=jax.ShapeDtypeStruct((1, 1), jnp.float32),
    )(tgt3, rows3)
    return out[0, 0]


def kernel(x, target):
    b, l, v = x.shape
    r = b * l
    block_rows = 512
    block_vocab = 8192
    x2 = x.reshape(r, v)
    tflat = target.reshape(r)
    tgt3 = tflat.reshape(r // block_rows, 1, block_rows)
    grows = _sc_gather_target_rows(x2, tflat)
    s1 = _tc_rowsum_loss(x2, tgt3, v, block_rows, block_vocab)
    s2 = _tc_gather_loss(tgt3, grows.reshape(r // block_rows, block_rows, 128), v, block_rows)
    return s1 + s2


# speedup vs baseline: 1.3473x; 1.0109x over previous
import functools

import jax
import jax.numpy as jnp
from jax import lax
from jax.experimental import pallas as pl
from jax.experimental.pallas import tpu as pltpu
from jax.experimental.pallas import tpu_sc as plsc

_PADDING_IDX = 0
_SMOOTHING = 0.1
_CONFIDENCE = 1.0 - _SMOOTHING


def _sc_gather_target_rows(x2, target_flat):
    n = target_flat.shape[0]
    info = plsc.get_sparse_core_info()
    nc, ns = info.num_cores, info.num_subcores
    nw = nc * ns
    n_per_w = n // nw
    mesh = plsc.VectorSubcoreMesh(core_axis_name="c", subcore_axis_name="s")
    chunk = 32

    @functools.partial(
        pl.kernel,
        mesh=mesh,
        out_type=jax.ShapeDtypeStruct((n, 128), jnp.float32),
        scratch_types=[
            pltpu.VMEM((n_per_w,), jnp.int32),
            pltpu.VMEM((2, chunk, 8, 128), jnp.float32),
            pltpu.VMEM((n_per_w, 128), jnp.float32),
            pltpu.SemaphoreType.DMA,
            pltpu.SemaphoreType.DMA,
        ],
    )
    def gather_kernel(x_hbm, tgt_hbm, out_hbm, t_v, tiles_v, rows_v, sem0, sem1):
        wid = lax.axis_index("s") * nc + lax.axis_index("c")
        base = wid * n_per_w
        pltpu.sync_copy(tgt_hbm.at[pl.ds(base, n_per_w)], t_v)
        sems = (sem0, sem1)

        def fire(k):
            copies = []
            for jo in range(chunk // 16):
                t_vec = t_v[pl.ds(k * chunk + jo * 16, 16)]
                cbs = lax.shift_left(lax.shift_right_logical(t_vec, 7), 7)
                for ji in range(16):
                    j = k * chunk + jo * 16 + ji
                    copies.append(
                        pltpu.async_copy(
                            x_hbm.at[
                                pl.ds(base + (j & ~7), 8),
                                pl.ds(pl.multiple_of(cbs[ji], 128), 128),
                            ],
                            tiles_v.at[k & 1, j - k * chunk],
                            sems[k & 1],
                        )
                    )
            return copies

        pending = fire(0)
        for k in range(n_per_w // chunk):
            nxt = fire(k + 1) if (k + 1) < n_per_w // chunk else []
            for c in pending:
                c.wait()
            pending = nxt
            for jj in range(chunk):
                j = k * chunk + jj
                for c8 in range(8):
                    rows_v[j, pl.ds(c8 * 16, 16)] = tiles_v[k & 1, jj, j & 7, pl.ds(c8 * 16, 16)]
        pltpu.sync_copy(rows_v, out_hbm.at[pl.ds(base, n_per_w)])

    return gather_kernel(x2, target_flat)


def _tc_rowsum_loss(x2, tgt3, vocab, block_rows, nbuf):
    rows = x2.shape[0]
    nblk = rows // block_rows
    e_k = _SMOOTHING / (vocab - 2)

    def body(x_hbm, t_ref, out_ref, bufs, sems):
        def start(blk, slot):
            pltpu.make_async_copy(
                x_hbm.at[pl.ds(blk * block_rows, block_rows), :],
                bufs.at[slot],
                sems.at[slot],
            ).start()

        def wait(slot):
            pltpu.make_async_copy(
                x_hbm.at[pl.ds(0, block_rows), :], bufs.at[slot], sems.at[slot]
            ).wait()

        for slot in range(nbuf):
            start(slot, slot)

        def step(i, acc):
            slot = lax.rem(i, nbuf)
            wait(slot)
            rs = jnp.sum(bufs[slot], axis=1)
            t = t_ref[i, 0, :]
            acc += jnp.sum(jnp.where(t != _PADDING_IDX, rs, 0.0))

            @pl.when(i + nbuf < nblk)
            def _():
                start(i + nbuf, slot)

            return acc

        acc = lax.fori_loop(0, nblk, step, jnp.float32(0.0))
        out_ref[...] = jnp.reshape(-e_k * acc, (1, 1))

    out = pl.pallas_call(
        body,
        in_specs=[
            pl.BlockSpec(memory_space=pl.ANY),
            pl.BlockSpec(memory_space=pltpu.VMEM),
        ],
        out_specs=pl.BlockSpec(memory_space=pltpu.VMEM),
        out_shape=jax.ShapeDtypeStruct((1, 1), jnp.float32),
        scratch_shapes=[
            pltpu.VMEM((nbuf, block_rows, vocab), jnp.float32),
            pltpu.SemaphoreType.DMA((nbuf,)),
        ],
    )(x2, tgt3)
    return out[0, 0]


def _tc_gather_loss(tgt2, rows3, vocab):
    e_k = _SMOOTHING / (vocab - 2)
    a, brows, _ = rows3.shape

    def body(t_ref, r_ref, out_ref):
        t = t_ref[...]
        gr = r_ref[...]
        c = lax.bitwise_and(t, 127)
        lane = lax.broadcasted_iota(jnp.int32, (a, brows, 128), 2)
        g = jnp.sum(jnp.where(lane == c[:, :, None], gr, 0.0), axis=2)
        per = jnp.where(t != _PADDING_IDX, g, 0.0)
        out_ref[...] = jnp.reshape(-(_CONFIDENCE - e_k) * jnp.sum(per), (1, 1))

    out = pl.pallas_call(
        body,
        in_specs=[
            pl.BlockSpec(memory_space=pltpu.VMEM),
            pl.BlockSpec(memory_space=pltpu.VMEM),
        ],
        out_specs=pl.BlockSpec(memory_space=pltpu.VMEM),
        out_shape=jax.ShapeDtypeStruct((1, 1), jnp.float32),
    )(tgt2, rows3)
    return out[0, 0]


def kernel(x, target):
    b, l, v = x.shape
    r = b * l
    block_rows = 128
    nbuf = 8
    x2 = x.reshape(r, v)
    tflat = target.reshape(r)
    tgt3 = tflat.reshape(r // block_rows, 1, block_rows)
    grows = _sc_gather_target_rows(x2, tflat)
    s1 = _tc_rowsum_loss(x2, tgt3, v, block_rows, nbuf)
    s2 = _tc_gather_loss(
        tflat.reshape(r // 128, 128), grows.reshape(r // 128, 128, 128), v
    )
    return s1 + s2


# speedup vs baseline: 1.3756x; 1.0209x over previous
import functools

import jax
import jax.numpy as jnp
from jax import lax
from jax.experimental import pallas as pl
from jax.experimental.pallas import tpu as pltpu
from jax.experimental.pallas import tpu_sc as plsc

_PADDING_IDX = 0
_SMOOTHING = 0.1
_CONFIDENCE = 1.0 - _SMOOTHING


def _sc_gather_target_rows(x2, target_flat):
    n = target_flat.shape[0]
    info = plsc.get_sparse_core_info()
    nc, ns = info.num_cores, info.num_subcores
    nw = nc * ns
    n_per_w = n // nw
    mesh = plsc.VectorSubcoreMesh(core_axis_name="c", subcore_axis_name="s")
    chunk = 32

    @functools.partial(
        pl.kernel,
        mesh=mesh,
        out_type=jax.ShapeDtypeStruct((n, 128), jnp.float32),
        scratch_types=[
            pltpu.VMEM((n_per_w,), jnp.int32),
            pltpu.VMEM((2, chunk, 8, 128), jnp.float32),
            pltpu.VMEM((n_per_w, 128), jnp.float32),
            pltpu.SemaphoreType.DMA,
            pltpu.SemaphoreType.DMA,
        ],
    )
    def gather_kernel(x_hbm, tgt_hbm, out_hbm, t_v, tiles_v, rows_v, sem0, sem1):
        wid = lax.axis_index("s") * nc + lax.axis_index("c")
        base = wid * n_per_w
        pltpu.sync_copy(tgt_hbm.at[pl.ds(base, n_per_w)], t_v)
        sems = (sem0, sem1)

        def fire(k):
            copies = []
            for jo in range(chunk // 16):
                t_vec = t_v[pl.ds(k * chunk + jo * 16, 16)]
                cbs = lax.shift_left(lax.shift_right_logical(t_vec, 7), 7)
                for ji in range(16):
                    j = k * chunk + jo * 16 + ji
                    copies.append(
                        pltpu.async_copy(
                            x_hbm.at[
                                pl.ds(base + (j & ~7), 8),
                                pl.ds(pl.multiple_of(cbs[ji], 128), 128),
                            ],
                            tiles_v.at[k & 1, j - k * chunk],
                            sems[k & 1],
                        )
                    )
            return copies

        pending = fire(0)
        for k in range(n_per_w // chunk):
            nxt = fire(k + 1) if (k + 1) < n_per_w // chunk else []
            for c in pending:
                c.wait()
            pending = nxt
            for jj in range(chunk):
                j = k * chunk + jj
                for c8 in range(8):
                    rows_v[j, pl.ds(c8 * 16, 16)] = tiles_v[k & 1, jj, j & 7, pl.ds(c8 * 16, 16)]
        pltpu.sync_copy(rows_v, out_hbm.at[pl.ds(base, n_per_w)])

    return gather_kernel(x2, target_flat)


def _tc_rowsum_loss(x2, tgt3, vocab, block_rows, nbuf):
    rows = x2.shape[0]
    nblk = rows // block_rows
    e_k = _SMOOTHING / (vocab - 2)

    def body(x_hbm, t_ref, out_ref, bufs, sems):
        def start(blk, slot):
            pltpu.make_async_copy(
                x_hbm.at[pl.ds(blk * block_rows, block_rows), :],
                bufs.at[slot],
                sems.at[slot],
            ).start()

        def wait(slot):
            pltpu.make_async_copy(
                x_hbm.at[pl.ds(0, block_rows), :], bufs.at[slot], sems.at[slot]
            ).wait()

        for slot in range(nbuf):
            start(slot, slot)

        def step(i, acc):
            slot = lax.rem(i, nbuf)
            wait(slot)
            rs = jnp.sum(bufs[slot], axis=1)
            t = t_ref[i, 0, :]
            acc += jnp.sum(jnp.where(t != _PADDING_IDX, rs, 0.0))

            @pl.when(i + nbuf < nblk)
            def _():
                start(i + nbuf, slot)

            return acc

        acc = lax.fori_loop(0, nblk, step, jnp.float32(0.0))
        out_ref[...] = jnp.reshape(-e_k * acc, (1, 1))

    out = pl.pallas_call(
        body,
        in_specs=[
            pl.BlockSpec(memory_space=pl.ANY),
            pl.BlockSpec(memory_space=pltpu.VMEM),
        ],
        out_specs=pl.BlockSpec(memory_space=pltpu.VMEM),
        out_shape=jax.ShapeDtypeStruct((1, 1), jnp.float32),
        scratch_shapes=[
            pltpu.VMEM((nbuf, block_rows, vocab), jnp.float32),
            pltpu.SemaphoreType.DMA((nbuf,)),
        ],
    )(x2, tgt3)
    return out[0, 0]


def _tc_gather_loss(tgt3, rows3, vocab, s1):
    e_k = _SMOOTHING / (vocab - 2)
    a, brows, _ = rows3.shape

    def body(t_ref, r_ref, s_ref, out_ref):
        t = t_ref[:, 0, :]
        gr = r_ref[...]
        c = lax.bitwise_and(t, 127)
        lane = lax.broadcasted_iota(jnp.int32, (a, brows, 128), 2)
        g = jnp.sum(jnp.where(lane == c[:, :, None], gr, 0.0), axis=2)
        per = jnp.where(t != _PADDING_IDX, g, 0.0)
        out_ref[...] = s_ref[...] + jnp.reshape(
            -(_CONFIDENCE - e_k) * jnp.sum(per), (1, 1)
        )

    out = pl.pallas_call(
        body,
        in_specs=[
            pl.BlockSpec(memory_space=pltpu.VMEM),
            pl.BlockSpec(memory_space=pltpu.VMEM),
            pl.BlockSpec(memory_space=pltpu.VMEM),
        ],
        out_specs=pl.BlockSpec(memory_space=pltpu.VMEM),
        out_shape=jax.ShapeDtypeStruct((1, 1), jnp.float32),
    )(tgt3, rows3, s1)
    return out[0, 0]


def kernel(x, target):
    b, l, v = x.shape
    r = b * l
    block_rows = 128
    nbuf = 8
    x2 = x.reshape(r, v)
    tflat = target.reshape(r)
    tgt3 = tflat.reshape(r // block_rows, 1, block_rows)
    grows = _sc_gather_target_rows(x2, tflat)
    s1 = _tc_rowsum_loss(x2, tgt3, v, block_rows, nbuf)
    return _tc_gather_loss(
        tgt3, grows.reshape(r // 128, 128, 128), v, jnp.reshape(s1, (1, 1))
    )
